# baseline (speedup 1.0000x reference)
import functools

import jax
import jax.numpy as jnp
from jax import lax
from jax.experimental import pallas as pl
from jax.experimental.pallas import tpu as pltpu
from jax.experimental.pallas import tpu_sc as plsc

_NB = 16
_LANES = 128


@functools.lru_cache(maxsize=None)
def _make_lookup(V, D, B):
    info = plsc.get_sparse_core_info()
    NC, NS = info.num_cores, info.num_subcores
    NW = NC * NS
    assert B % (8 * NW) == 0, (B, NW)
    b_per_w = B // NW
    n_groups = b_per_w // _NB
    assert b_per_w % _NB == 0
    mesh = plsc.VectorSubcoreMesh(core_axis_name="c", subcore_axis_name="s")

    @functools.partial(
        pl.kernel,
        mesh=mesh,
        compiler_params=pltpu.CompilerParams(needs_layout_passes=False),
        out_type=jax.ShapeDtypeStruct((D, B), jnp.float32),
        scratch_types=[
            pltpu.VMEM((b_per_w,), jnp.int32),
            pltpu.VMEM((_NB, D, _LANES), jnp.float32),
            pltpu.VMEM((D, b_per_w), jnp.float32),
            [pltpu.SemaphoreType.DMA] * _NB,
        ],
    )
    def k(tT_hbm, idx_hbm, out_hbm, idx_v, tbuf, cols_v, sems):
        wid = lax.axis_index("s") * NC + lax.axis_index("c")
        base = wid * b_per_w
        pltpu.sync_copy(idx_hbm.at[pl.ds(base, b_per_w)], idx_v)

        c_lo = lax.iota(jnp.int32, 16)
        c_hi = c_lo + 16

        def fetch(r, slot):
            t_off = pl.multiple_of((r >> 7) << 7, _LANES)
            for a in range(4):
                pltpu.async_copy(
                    tT_hbm.at[pl.ds(a * 8, 8), pl.ds(t_off, _LANES)],
                    tbuf.at[slot, pl.ds(a * 8, 8), :],
                    sems[slot],
                )

        def drain(slot):
            pltpu.make_async_copy(
                tT_hbm.at[:, pl.ds(0, _LANES)], tbuf.at[slot], sems[slot]
            ).wait()

        def select(r, slot, j):
            l_splat = jnp.full((16,), r & 127, jnp.int32)
            j_splat = jnp.full((16,), j, jnp.int32)
            lo = plsc.load_gather(tbuf.at[slot], [c_lo, l_splat])
            hi = plsc.load_gather(tbuf.at[slot], [c_hi, l_splat])
            plsc.store_scatter(cols_v, [c_lo, j_splat], lo)
            plsc.store_scatter(cols_v, [c_hi, j_splat], hi)

        rv0 = idx_v[pl.ds(0, _NB)]
        for s in range(_NB):
            fetch(rv0[s], s)

        @pl.loop(0, n_groups)
        def group(g):
            j0 = g * _NB
            rv = idx_v[pl.ds(j0, _NB)]
            rv_next = idx_v[pl.ds(jnp.minimum(j0 + _NB, b_per_w - _NB), _NB)]
            for s in range(_NB):
                drain(s)
                select(rv[s], s, j0 + s)

                @pl.when(g + 1 < n_groups)
                def _():
                    fetch(rv_next[s], s)

        pltpu.sync_copy(cols_v, out_hbm.at[:, pl.ds(base, b_per_w)])

    return k


def kernel(node_vecs, country_idx):
    V, D = node_vecs.shape
    B = country_idx.shape[0]
    idx = country_idx.reshape(B).astype(jnp.int32)
    outT = _make_lookup(V, D, B)(node_vecs.T, idx)
    return outT.T


# speedup vs baseline: 1.0039x; 1.0039x over previous
import functools

import jax
import jax.numpy as jnp
from jax import lax
from jax.experimental import pallas as pl
from jax.experimental.pallas import tpu as pltpu
from jax.experimental.pallas import tpu_sc as plsc

_NB = 16
_LANES = 128


@functools.lru_cache(maxsize=None)
def _make_lookup(V, D, B):
    info = plsc.get_sparse_core_info()
    NC, NS = info.num_cores, info.num_subcores
    NW = NC * NS
    assert B % (8 * NW) == 0, (B, NW)
    b_per_w = B // NW
    n_groups = b_per_w // _NB
    assert b_per_w % _NB == 0
    mesh = plsc.VectorSubcoreMesh(core_axis_name="c", subcore_axis_name="s")

    @functools.partial(
        pl.kernel,
        mesh=mesh,
        compiler_params=pltpu.CompilerParams(needs_layout_passes=False),
        out_type=jax.ShapeDtypeStruct((D, B), jnp.float32),
        scratch_types=[
            pltpu.VMEM((b_per_w,), jnp.int32),
            pltpu.VMEM((_NB, D, _LANES), jnp.float32),
            pltpu.VMEM((D, b_per_w), jnp.float32),
            [pltpu.SemaphoreType.DMA] * _NB,
        ],
    )
    def k(tT_hbm, idx_hbm, out_hbm, idx_v, tbuf, cols_v, sems):
        wid = lax.axis_index("s") * NC + lax.axis_index("c")
        base = wid * b_per_w
        pltpu.sync_copy(idx_hbm.at[pl.ds(base, b_per_w)], idx_v)

        c_lo = lax.iota(jnp.int32, 16)
        c_hi = c_lo + 16

        def fetch(r, slot):
            t_off = pl.multiple_of((r >> 7) << 7, _LANES)
            pltpu.async_copy(
                tT_hbm.at[:, pl.ds(t_off, _LANES)], tbuf.at[slot], sems[slot]
            )

        def drain(slot):
            pltpu.make_async_copy(
                tT_hbm.at[:, pl.ds(0, _LANES)], tbuf.at[slot], sems[slot]
            ).wait()

        def select(r, slot, j):
            l_splat = jnp.full((16,), r & 127, jnp.int32)
            j_splat = jnp.full((16,), j, jnp.int32)
            lo = plsc.load_gather(tbuf.at[slot], [c_lo, l_splat])
            hi = plsc.load_gather(tbuf.at[slot], [c_hi, l_splat])
            plsc.store_scatter(cols_v, [c_lo, j_splat], lo)
            plsc.store_scatter(cols_v, [c_hi, j_splat], hi)

        rv0 = idx_v[pl.ds(0, _NB)]
        for s in range(_NB):
            fetch(rv0[s], s)

        @pl.loop(0, n_groups)
        def group(g):
            j0 = g * _NB
            rv = idx_v[pl.ds(j0, _NB)]
            rv_next = idx_v[pl.ds(jnp.minimum(j0 + _NB, b_per_w - _NB), _NB)]
            for s in range(_NB):
                drain(s)
                select(rv[s], s, j0 + s)

                @pl.when(g + 1 < n_groups)
                def _():
                    fetch(rv_next[s], s)

        pltpu.sync_copy(cols_v, out_hbm.at[:, pl.ds(base, b_per_w)])

    return k


def kernel(node_vecs, country_idx):
    V, D = node_vecs.shape
    B = country_idx.shape[0]
    idx = country_idx.reshape(B).astype(jnp.int32)
    outT = _make_lookup(V, D, B)(node_vecs.T, idx)
    return outT.T
